# Initial kernel scaffold; baseline (speedup 1.0000x reference)
import jax
import jax.numpy as jnp
from jax.experimental import pallas as pl


def kernel(documents, queries, query_ids, W):
    raise NotImplementedError("write your pallas kernel here")


# speedup vs baseline: 9.2793x; 9.2793x over previous
import functools

import jax
import jax.numpy as jnp
from jax import lax
from jax.experimental import pallas as pl
from jax.experimental.pallas import tpu as pltpu
from jax.experimental.pallas import tpu_sc as plsc

NC, NS = 2, 16
NW = NC * NS
B = 4096
LD, LQ = 200, 20
D = 128
DC = D // 16
BW = B // NW
CA, CB = 128, LD - 128
QP = 2


def _rsqrt(x):
    i = plsc.bitcast(x, jnp.int32)
    i = jnp.full((16,), 0x5F3759DF, jnp.int32) - lax.shift_right_logical(i, 1)
    y = plsc.bitcast(i, jnp.float32)
    for _ in range(3):
        y = y * (1.5 - 0.5 * x * y * y)
    return y


def kernel(documents, queries, query_ids, W):
    del query_ids
    doc_flat = documents.reshape(-1)
    q_flat = queries.reshape(-1)

    mesh = plsc.VectorSubcoreMesh(
        core_axis_name="c", subcore_axis_name="s", num_cores=NC, num_subcores=NS
    )

    @functools.partial(
        pl.kernel,
        out_type=jax.ShapeDtypeStruct((B,), jnp.float32),
        mesh=mesh,
        compiler_params=pltpu.CompilerParams(needs_layout_passes=False),
        scratch_types=[
            pltpu.VMEM((BW * LD,), jnp.int32),
            pltpu.VMEM((BW * LQ,), jnp.int32),
            pltpu.VMEM((LD, D), jnp.float32),
            pltpu.VMEM((LD, D), jnp.float32),
            pltpu.VMEM((QP * LQ, D), jnp.float32),
            pltpu.VMEM((QP * LQ, D), jnp.float32),
            pltpu.VMEM((BW, D), jnp.float32),
            pltpu.VMEM((BW, D), jnp.float32),
            pltpu.VMEM((BW,), jnp.float32),
            pltpu.SemaphoreType.DMA,
            pltpu.SemaphoreType.DMA,
        ],
    )
    def sc_kernel(doc_hbm, q_hbm, w_hbm, out_hbm,
                  didx, qidx, dbufa, dbufb, qbufa, qbufb,
                  dsum, qsum, res, sema, semb):
        wid = lax.axis_index("s") * NC + lax.axis_index("c")
        base = wid * BW

        pltpu.sync_copy(doc_hbm.at[pl.ds(base * LD, BW * LD)], didx)
        pltpu.sync_copy(q_hbm.at[pl.ds(base * LQ, BW * LQ)], qidx)

        def d_issue(j, buf):
            pltpu.make_async_copy(
                w_hbm.at[didx.at[pl.ds(j * LD, CA)]],
                buf.at[pl.ds(0, CA)], sema if buf is dbufa else semb,
            ).start()
            pltpu.make_async_copy(
                w_hbm.at[didx.at[pl.ds(j * LD + CA, CB)]],
                buf.at[pl.ds(CA, CB)], sema if buf is dbufa else semb,
            ).start()

        def d_wait(j, buf):
            sem = sema if buf is dbufa else semb
            pltpu.make_async_copy(
                w_hbm.at[didx.at[pl.ds(j * LD, CA)]],
                buf.at[pl.ds(0, CA)], sem).wait()
            pltpu.make_async_copy(
                w_hbm.at[didx.at[pl.ds(j * LD + CA, CB)]],
                buf.at[pl.ds(CA, CB)], sem).wait()

        def reduce_rows(buf, lo, n, out_ref, orow):
            def body(r, accs):
                return tuple(
                    accs[c] + buf[r, pl.ds(c * 16, 16)] for c in range(DC)
                )
            zero = jnp.zeros((16,), jnp.float32)
            accs = lax.fori_loop(lo, lo + n, body, (zero,) * DC)
            for c in range(DC):
                out_ref[orow, pl.ds(c * 16, 16)] = accs[c]

        d_issue(0, dbufa)

        def d_body(j2, _):
            j = 2 * j2
            d_wait(j, dbufa)
            d_issue(j + 1, dbufb)
            reduce_rows(dbufa, 0, LD, dsum, j)

            d_wait(j + 1, dbufb)

            @pl.when(j + 2 < BW)
            def _():
                d_issue(j + 2, dbufa)

            reduce_rows(dbufb, 0, LD, dsum, j + 1)
            return 0

        lax.fori_loop(0, BW // 2, d_body, 0)

        def q_issue(p, buf):
            pltpu.make_async_copy(
                w_hbm.at[qidx.at[pl.ds(p * QP * LQ, QP * LQ)]],
                buf, sema if buf is qbufa else semb,
            ).start()

        def q_wait(p, buf):
            pltpu.make_async_copy(
                w_hbm.at[qidx.at[pl.ds(p * QP * LQ, QP * LQ)]],
                buf, sema if buf is qbufa else semb,
            ).wait()

        NP = BW // QP
        q_issue(0, qbufa)

        def q_body(p2, _):
            p = 2 * p2
            q_wait(p, qbufa)
            q_issue(p + 1, qbufb)
            reduce_rows(qbufa, 0, LQ, qsum, p * QP)
            reduce_rows(qbufa, LQ, LQ, qsum, p * QP + 1)

            q_wait(p + 1, qbufb)

            @pl.when(p + 2 < NP)
            def _():
                q_issue(p + 2, qbufa)

            reduce_rows(qbufb, 0, LQ, qsum, (p + 1) * QP)
            reduce_rows(qbufb, LQ, LQ, qsum, (p + 1) * QP + 1)
            return 0

        lax.fori_loop(0, NP // 2, q_body, 0)

        eps = 1e-8
        dscale = 1.0 / LD
        qscale = 1.0 / LQ
        lanes = lax.iota(jnp.int32, 16)

        def cos_group(g, _):
            dotv = jnp.zeros((16,), jnp.float32)
            ddv = jnp.zeros((16,), jnp.float32)
            qqv = jnp.zeros((16,), jnp.float32)
            for l in range(16):
                row = g * 16 + l
                pd = jnp.zeros((16,), jnp.float32)
                dd = jnp.zeros((16,), jnp.float32)
                qq = jnp.zeros((16,), jnp.float32)
                for c in range(DC):
                    dv = dsum[row, pl.ds(c * 16, 16)] * dscale
                    qv = qsum[row, pl.ds(c * 16, 16)] * qscale
                    pd = pd + dv * qv
                    dd = dd + dv * dv
                    qq = qq + qv * qv
                sel = lanes == l
                dotv = jnp.where(sel, jnp.sum(pd), dotv)
                ddv = jnp.where(sel, jnp.sum(dd), ddv)
                qqv = jnp.where(sel, jnp.sum(qq), qqv)
            dn = jnp.maximum(ddv * _rsqrt(ddv), eps)
            qn = jnp.maximum(qqv * _rsqrt(qqv), eps)
            res[pl.ds(g * 16, 16)] = dotv / (dn * qn)
            return 0

        lax.fori_loop(0, BW // 16, cos_group, 0)

        pltpu.sync_copy(res, out_hbm.at[pl.ds(base, BW)])

    return sc_kernel(doc_flat, q_flat, W)
